# baseline (speedup 1.0000x reference)
import functools

import jax
import jax.numpy as jnp
from jax import lax
from jax.experimental import pallas as pl
from jax.experimental.pallas import tpu as pltpu
from jax.experimental.pallas import tpu_sc as plsc

N_NODES = 50000
N_EDGES = 1600000
N_GRAPHS = 1000
D = 128
DE = 16
CN = 80
N_NODE_CHUNKS = N_NODES // CN
CE = 80
N_EDGE_CHUNKS = N_EDGES // CE

NC = 2
NS = 16
NW = NC * NS

EC_PER_W = N_EDGE_CHUNKS // NW
NGV = (N_GRAPHS + 15) // 16

OFF_CHARGE = 100
OFF_TIME = 113
OFF_COUNT = 1113
NT_ROWS = 1369

B_SENT = N_NODES
B_TIME = N_NODES + 8
B_CNT = N_NODES + 8 + N_GRAPHS
BIG = B_CNT + NGV * 16

_mesh = plsc.VectorSubcoreMesh(core_axis_name="c", subcore_axis_name="s",
                               num_cores=NC, num_subcores=NS)


def _backbone_body(ints_hbm, nidx_hbm, e_hbm, wn_hbm, we_hbm,
                   h0_out, ee_out,
                   big, idx6, rows, e_rows, sem):
    wid = lax.axis_index("s") * NC + lax.axis_index("c")

    pltpu.sync_copy(ints_hbm, big.at[pl.ds(0, B_CNT)])

    lanes = lax.iota(jnp.int32, 16)

    def lower_bound(tgt):
        lo = jnp.zeros((16,), jnp.int32)
        hi = jnp.full((16,), N_NODES, jnp.int32)
        for _ in range(16):
            mid = (lo + hi) >> 1
            vals = plsc.load_gather(big, [mid])
            lt = vals < tgt
            lo = jnp.where(lt, mid + 1, lo)
            hi = jnp.where(lt, hi, mid)
        return lo

    def cnt_body(k, carry):
        g = k * 16 + lanes
        cnt = lower_bound(g + 1) - lower_bound(g)
        cnt = jnp.clip(cnt, 0, 255)
        big[pl.ds(B_CNT + k * 16, 16)] = cnt + OFF_COUNT
        return carry

    lax.fori_loop(0, NGV, cnt_body, 0)

    lo = wid * N_NODE_CHUNKS // NW
    hi = (wid + 1) * N_NODE_CHUNKS // NW

    def node_body(chunk, carry):
        base = chunk * CN
        pltpu.sync_copy(nidx_hbm.at[chunk], idx6.at[pl.ds(2 * CN, 3 * CN)])
        for v in range(CN // 16):
            bv = idx6[pl.ds(2 * CN + v * 16, 16)]
            idx6[pl.ds(5 * CN + v * 16, 16)] = plsc.load_gather(big, [bv + B_CNT])
            idx6[pl.ds(6 * CN + v * 16, 16)] = plsc.load_gather(big, [bv + B_TIME])
        cps = [
            pltpu.async_copy(wn_hbm.at[idx6.at[pl.ds((3 + b) * CN, CN)]],
                             rows.at[b], sem)
            for b in range(4)
        ]
        for cp in cps:
            cp.wait()
        for b in range(4):
            pltpu.sync_copy(rows.at[b],
                            h0_out.at[pl.ds(base, CN), pl.ds(b * D, D)])
        return carry

    lax.fori_loop(lo, hi, node_body, 0)

    def edge_body(k, carry):
        base = k * CE
        pltpu.sync_copy(e_hbm.at[pl.ds(base, CE)], idx6.at[pl.ds(0, CE)])
        pltpu.async_copy(we_hbm.at[idx6.at[pl.ds(0, CE)]], e_rows, sem).wait()
        pltpu.sync_copy(e_rows, ee_out.at[pl.ds(base, CE)])
        return carry

    lax.fori_loop(wid * EC_PER_W, (wid + 1) * EC_PER_W, edge_body, 0)


_backbone = functools.partial(
    pl.kernel,
    out_type=(
        jax.ShapeDtypeStruct((N_NODES, 4 * D), jnp.float32),
        jax.ShapeDtypeStruct((N_EDGES, DE), jnp.float32),
    ),
    mesh=_mesh,
    compiler_params=pltpu.CompilerParams(needs_layout_passes=False,
                                         use_tc_tiling_on_sc=False),
    scratch_types=[
        pltpu.VMEM((BIG,), jnp.int32),
        pltpu.VMEM((7 * CN,), jnp.int32),
        pltpu.VMEM((4, CN, D), jnp.float32),
        pltpu.VMEM((CE, DE), jnp.float32),
        pltpu.SemaphoreType.DMA,
    ],
)(_backbone_body)


def kernel(a, c, e, edge_index, t, batch, W_atom, W_charge, W_edge, W_time, W_count):
    a = jnp.asarray(a, jnp.int32)
    c = jnp.asarray(c, jnp.int32)
    e = jnp.asarray(e, jnp.int32)
    t = jnp.asarray(t, jnp.int32)
    batch = jnp.asarray(batch, jnp.int32)
    wn = jnp.concatenate([W_atom, W_charge, W_time, W_count], axis=0)
    sent = jnp.full((8,), jnp.iinfo(jnp.int32).max, jnp.int32)
    ints = jnp.concatenate([batch, sent, t + OFF_TIME])
    nidx = jnp.stack([batch.reshape(N_NODE_CHUNKS, CN),
                      a.reshape(N_NODE_CHUNKS, CN),
                      c.reshape(N_NODE_CHUNKS, CN) + OFF_CHARGE],
                     axis=1).reshape(N_NODE_CHUNKS, 3 * CN)
    h0, e_embed = _backbone(ints, nidx, e, wn, W_edge)
    return (h0, (edge_index[0], edge_index[1]), e_embed)


# speedup vs baseline: 1.0028x; 1.0028x over previous
import functools

import jax
import jax.numpy as jnp
from jax import lax
from jax.experimental import pallas as pl
from jax.experimental.pallas import tpu as pltpu
from jax.experimental.pallas import tpu_sc as plsc

N_NODES = 50000
N_EDGES = 1600000
N_GRAPHS = 1000
D = 128
DE = 16
CN = 80
N_NODE_CHUNKS = N_NODES // CN
CE = 80
EB = 400

NC = 2
NS = 16
NW = NC * NS

EPT = N_EDGES // NW
EBLK = EPT // EB
NGV = (N_GRAPHS + 15) // 16

OFF_CHARGE = 100
OFF_TIME = 113
OFF_COUNT = 1113
NT_ROWS = 1369

B_SENT = N_NODES
B_TIME = N_NODES + 8
B_CNT = N_NODES + 8 + N_GRAPHS
BIG = B_CNT + NGV * 16

_mesh = plsc.VectorSubcoreMesh(core_axis_name="c", subcore_axis_name="s",
                               num_cores=NC, num_subcores=NS)


def _backbone_body(ints_hbm, nidx_hbm, e_hbm, wn_hbm, we_hbm,
                   h0_out, ee_out,
                   big, idx6, rows, e_rows, sem, semw):
    wid = lax.axis_index("s") * NC + lax.axis_index("c")

    pltpu.sync_copy(ints_hbm, big.at[pl.ds(0, B_CNT)])

    lanes = lax.iota(jnp.int32, 16)

    def lower_bound(tgt):
        lo = jnp.zeros((16,), jnp.int32)
        hi = jnp.full((16,), N_NODES, jnp.int32)
        for _ in range(16):
            mid = (lo + hi) >> 1
            vals = plsc.load_gather(big, [mid])
            lt = vals < tgt
            lo = jnp.where(lt, mid + 1, lo)
            hi = jnp.where(lt, hi, mid)
        return lo

    def cnt_body(k, carry):
        g = k * 16 + lanes
        cnt = lower_bound(g + 1) - lower_bound(g)
        cnt = jnp.clip(cnt, 0, 255)
        big[pl.ds(B_CNT + k * 16, 16)] = cnt + OFF_COUNT
        return carry

    lax.fori_loop(0, NGV, cnt_body, 0)

    lo = wid * N_NODE_CHUNKS // NW
    hi = (wid + 1) * N_NODE_CHUNKS // NW

    def node_body(chunk, carry):
        base = chunk * CN
        pltpu.sync_copy(nidx_hbm.at[chunk], idx6.at[pl.ds(2 * CN, 3 * CN)])
        for v in range(CN // 16):
            bv = idx6[pl.ds(2 * CN + v * 16, 16)]
            idx6[pl.ds(5 * CN + v * 16, 16)] = plsc.load_gather(big, [bv + B_CNT])
            idx6[pl.ds(6 * CN + v * 16, 16)] = plsc.load_gather(big, [bv + B_TIME])
        cps = [
            pltpu.async_copy(wn_hbm.at[idx6.at[pl.ds((3 + b) * CN, CN)]],
                             rows.at[b], sem)
            for b in range(4)
        ]
        for cp in cps:
            cp.wait()
        for b in range(4):
            pltpu.sync_copy(rows.at[b],
                            h0_out.at[pl.ds(base, CN), pl.ds(b * D, D)])
        return carry

    lax.fori_loop(lo, hi, node_body, 0)

    ebase = wid * EPT
    pltpu.sync_copy(e_hbm.at[pl.ds(ebase, EPT)], big.at[pl.ds(0, EPT)])

    def fire_gathers(off, buf):
        cps = [pltpu.async_copy(
                   we_hbm.at[big.at[pl.ds(off + j * CE, CE)]],
                   e_rows.at[buf, pl.ds(j * CE, CE)], sem)
               for j in range(EB // CE)]
        for cp in cps:
            cp.wait()

    def fire_write(off, buf):
        pltpu.async_copy(e_rows.at[buf], ee_out.at[pl.ds(ebase + off, EB)],
                         semw)

    def wait_write(off, buf):
        pltpu.make_async_copy(e_rows.at[buf],
                              ee_out.at[pl.ds(ebase + off, EB)], semw).wait()

    fire_gathers(0, 0)
    fire_write(0, 0)
    fire_gathers(EB, 1)
    wait_write(0, 0)
    fire_write(EB, 1)

    def pair_body(p, carry):
        o0 = 2 * p * EB
        o1 = o0 + EB
        fire_gathers(o0, 0)
        wait_write(o1 - 2 * EB, 1)
        fire_write(o0, 0)
        fire_gathers(o1, 1)
        wait_write(o0, 0)
        fire_write(o1, 1)
        return carry

    lax.fori_loop(1, EBLK // 2, pair_body, 0)

    o_last = (EBLK - 1) * EB
    fire_gathers(o_last, 0)
    wait_write(o_last - EB, 1)
    fire_write(o_last, 0)
    wait_write(o_last, 0)


_backbone = functools.partial(
    pl.kernel,
    out_type=(
        jax.ShapeDtypeStruct((N_NODES, 4 * D), jnp.float32),
        jax.ShapeDtypeStruct((N_EDGES, DE), jnp.float32),
    ),
    mesh=_mesh,
    compiler_params=pltpu.CompilerParams(needs_layout_passes=False,
                                         use_tc_tiling_on_sc=False),
    scratch_types=[
        pltpu.VMEM((BIG,), jnp.int32),
        pltpu.VMEM((7 * CN,), jnp.int32),
        pltpu.VMEM((4, CN, D), jnp.float32),
        pltpu.VMEM((2, EB, DE), jnp.float32),
        pltpu.SemaphoreType.DMA,
        pltpu.SemaphoreType.DMA,
    ],
)(_backbone_body)


def kernel(a, c, e, edge_index, t, batch, W_atom, W_charge, W_edge, W_time, W_count):
    a = jnp.asarray(a, jnp.int32)
    c = jnp.asarray(c, jnp.int32)
    e = jnp.asarray(e, jnp.int32)
    t = jnp.asarray(t, jnp.int32)
    batch = jnp.asarray(batch, jnp.int32)
    wn = jnp.concatenate([W_atom, W_charge, W_time, W_count], axis=0)
    sent = jnp.full((8,), jnp.iinfo(jnp.int32).max, jnp.int32)
    ints = jnp.concatenate([batch, sent, t + OFF_TIME])
    nidx = jnp.stack([batch.reshape(N_NODE_CHUNKS, CN),
                      a.reshape(N_NODE_CHUNKS, CN),
                      c.reshape(N_NODE_CHUNKS, CN) + OFF_CHARGE],
                     axis=1).reshape(N_NODE_CHUNKS, 3 * CN)
    h0, e_embed = _backbone(ints, nidx, e, wn, W_edge)
    return (h0, (edge_index[0], edge_index[1]), e_embed)


# speedup vs baseline: 9.6406x; 9.6136x over previous
import functools

import jax
import jax.numpy as jnp
from jax import lax
from jax.experimental import pallas as pl
from jax.experimental.pallas import tpu as pltpu
from jax.experimental.pallas import tpu_sc as plsc

N_NODES = 50000
N_EDGES = 1600000
N_GRAPHS = 1000
D = 128
DE = 16
CN = 80
N_NODE_CHUNKS = N_NODES // CN
CE = 80
EB = 400

NC = 2
NS = 16
NW = NC * NS

EPT = N_EDGES // NW
EBLK = EPT // EB
NGV = (N_GRAPHS + 15) // 16

OFF_CHARGE = 100
OFF_TIME = 113
OFF_COUNT = 1113
NT_ROWS = 1369

B_SENT = N_NODES
B_TIME = N_NODES + 8
B_CNT = N_NODES + 8 + N_GRAPHS
BIG = B_CNT + NGV * 16

_mesh = plsc.VectorSubcoreMesh(core_axis_name="c", subcore_axis_name="s",
                               num_cores=NC, num_subcores=NS)


def _take16(vec, idx):
    dnums = lax.GatherDimensionNumbers(offset_dims=(),
                                       collapsed_slice_dims=(0,),
                                       start_index_map=(0,))
    return lax.gather(vec, idx[:, None], dnums, (1,),
                      mode=lax.GatherScatterMode.PROMISE_IN_BOUNDS)


def _backbone_body(ints_hbm, nidx_hbm, e_hbm, wn_hbm, wet_hbm,
                   h0_out, ee_out,
                   big, idx6, rows, e_rows, wn_sp, wet_v, sem, semw):
    wid = lax.axis_index("s") * NC + lax.axis_index("c")

    stage = pltpu.async_copy(wn_hbm, wn_sp, semw)
    pltpu.sync_copy(wet_hbm, wet_v)
    pltpu.sync_copy(ints_hbm, big.at[pl.ds(0, B_CNT)])

    lanes = lax.iota(jnp.int32, 16)

    def lower_bound(tgt):
        lo = jnp.zeros((16,), jnp.int32)
        hi = jnp.full((16,), N_NODES, jnp.int32)
        for _ in range(16):
            mid = (lo + hi) >> 1
            vals = plsc.load_gather(big, [mid])
            lt = vals < tgt
            lo = jnp.where(lt, mid + 1, lo)
            hi = jnp.where(lt, hi, mid)
        return lo

    def cnt_body(k, carry):
        g = k * 16 + lanes
        cnt = lower_bound(g + 1) - lower_bound(g)
        cnt = jnp.clip(cnt, 0, 255)
        big[pl.ds(B_CNT + k * 16, 16)] = cnt + OFF_COUNT
        return carry

    lax.fori_loop(0, NGV, cnt_body, 0)
    stage.wait()

    lo = wid * N_NODE_CHUNKS // NW
    hi = (wid + 1) * N_NODE_CHUNKS // NW

    def wait_node_write():
        pltpu.make_async_copy(rows.at[0],
                              h0_out.at[pl.ds(0, CN), pl.ds(0, D)],
                              semw).wait()

    def node_chunk(chunk, first):
        base = chunk * CN
        pltpu.sync_copy(nidx_hbm.at[chunk], idx6.at[pl.ds(2 * CN, 3 * CN)])
        for v in range(CN // 16):
            bv = idx6[pl.ds(2 * CN + v * 16, 16)]
            idx6[pl.ds(5 * CN + v * 16, 16)] = plsc.load_gather(big, [bv + B_CNT])
            idx6[pl.ds(6 * CN + v * 16, 16)] = plsc.load_gather(big, [bv + B_TIME])
        if not first:
            for _ in range(4):
                wait_node_write()
        cps = [pltpu.async_copy(wn_sp.at[idx6.at[pl.ds((3 + b) * CN, CN)]],
                                rows.at[b], sem)
               for b in range(4)]
        for cp in cps:
            cp.wait()
        for b in range(4):
            pltpu.async_copy(rows.at[b],
                             h0_out.at[pl.ds(base, CN), pl.ds(b * D, D)],
                             semw)

    node_chunk(lo, True)

    def node_body(chunk, carry):
        node_chunk(chunk, False)
        return carry

    lax.fori_loop(lo + 1, hi, node_body, 0)
    for _ in range(4):
        wait_node_write()

    ebase = wid * EPT
    pltpu.sync_copy(e_hbm.at[pl.ds(ebase, EPT)], big.at[pl.ds(0, EPT)])
    wcols = [wet_v[pl.ds(k * 16, 16)] for k in range(DE)]
    scat0 = lanes * DE

    def compute_block(off, buf):
        bbase = buf * (EB * DE)

        def group(g, carry):
            ev = big[pl.ds(off + g * 16, 16)]
            sbase = bbase + g * (16 * DE)
            for k in range(DE):
                col = _take16(wcols[k], ev)
                plsc.store_scatter(e_rows, [scat0 + (sbase + k)], col)
            return carry

        lax.fori_loop(0, EB // 16, group, 0)

    def fire_write(off, buf):
        pltpu.async_copy(e_rows.at[pl.ds(buf * EB * DE, EB * DE)],
                         ee_out.at[pl.ds((ebase + off) * DE, EB * DE)],
                         semw)

    def wait_write(off, buf):
        pltpu.make_async_copy(e_rows.at[pl.ds(buf * EB * DE, EB * DE)],
                              ee_out.at[pl.ds((ebase + off) * DE, EB * DE)],
                              semw).wait()

    compute_block(0, 0)
    fire_write(0, 0)
    compute_block(EB, 1)
    wait_write(0, 0)
    fire_write(EB, 1)

    def pair_body(p, carry):
        o0 = 2 * p * EB
        o1 = o0 + EB
        compute_block(o0, 0)
        wait_write(o1 - 2 * EB, 1)
        fire_write(o0, 0)
        compute_block(o1, 1)
        wait_write(o0, 0)
        fire_write(o1, 1)
        return carry

    lax.fori_loop(1, EBLK // 2, pair_body, 0)

    o_last = (EBLK - 1) * EB
    compute_block(o_last, 0)
    wait_write(o_last - EB, 1)
    fire_write(o_last, 0)
    wait_write(o_last, 0)


_backbone = functools.partial(
    pl.kernel,
    out_type=(
        jax.ShapeDtypeStruct((N_NODES, 4 * D), jnp.float32),
        jax.ShapeDtypeStruct((N_EDGES * DE,), jnp.float32),
    ),
    mesh=_mesh,
    compiler_params=pltpu.CompilerParams(needs_layout_passes=False,
                                         use_tc_tiling_on_sc=False),
    scratch_types=[
        pltpu.VMEM((BIG,), jnp.int32),
        pltpu.VMEM((7 * CN,), jnp.int32),
        pltpu.VMEM((4, CN, D), jnp.float32),
        pltpu.VMEM((2 * EB * DE,), jnp.float32),
        pltpu.VMEM_SHARED((NT_ROWS, D), jnp.float32),
        pltpu.VMEM((DE * 16,), jnp.float32),
        pltpu.SemaphoreType.DMA,
        pltpu.SemaphoreType.DMA,
    ],
)(_backbone_body)


def kernel(a, c, e, edge_index, t, batch, W_atom, W_charge, W_edge, W_time, W_count):
    a = jnp.asarray(a, jnp.int32)
    c = jnp.asarray(c, jnp.int32)
    e = jnp.asarray(e, jnp.int32)
    t = jnp.asarray(t, jnp.int32)
    batch = jnp.asarray(batch, jnp.int32)
    wn = jnp.concatenate([W_atom, W_charge, W_time, W_count], axis=0)
    sent = jnp.full((8,), jnp.iinfo(jnp.int32).max, jnp.int32)
    ints = jnp.concatenate([batch, sent, t + OFF_TIME])
    nidx = jnp.stack([batch.reshape(N_NODE_CHUNKS, CN),
                      a.reshape(N_NODE_CHUNKS, CN),
                      c.reshape(N_NODE_CHUNKS, CN) + OFF_CHARGE],
                     axis=1).reshape(N_NODE_CHUNKS, 3 * CN)
    wet = jnp.zeros((DE, 16), jnp.float32).at[:, :5].set(W_edge.T).reshape(-1)
    h0, e_flat = _backbone(ints, nidx, e, wn, wet)
    return (h0, (edge_index[0], edge_index[1]), e_flat.reshape(N_EDGES, DE))
